# baseline (speedup 1.0000x reference)
import functools

import jax
import jax.numpy as jnp
from jax import lax
from jax.experimental import pallas as pl
from jax.experimental.pallas import tpu as pltpu
from jax.experimental.pallas import tpu_sc as plsc

_NE = 8192
_D = 256
_NV = 16384

_BZ = 1024
_BC = 2048
_NZB = _NV // _BZ
_NCB = _NE // _BC


_LANES = 128
_KS = _BC // _LANES
_BR = 64
_NRB = _BZ // _BR


def _dist_argmin_body(a2_ref, b2_ref, zb_ref, cbt_ref, idx_ref, loss_ref,
                      pmin_ref, pblk_ref, ab_ref):
    i = pl.program_id(0)
    j = pl.program_id(1)

    @pl.when(j == 0)
    def _():
        pmin_ref[...] = jnp.full((_BZ, _LANES), jnp.inf, jnp.float32)

    ab_ref[...] = lax.dot_general(
        zb_ref[...] * 2.0, cbt_ref[...], (((1,), (0,)), ((), ())),
        preferred_element_type=jnp.float32)
    b2 = b2_ref[...]

    def row_block(r, _):
        rs = pl.ds(pl.multiple_of(r * _BR, _BR), _BR)
        a2r = a2_ref[rs, :]
        pm = pmin_ref[rs]
        pb = pblk_ref[rs]
        for k in range(_KS):
            lo, hi = k * _LANES, (k + 1) * _LANES
            t1 = a2r + b2[lo:hi][None, :]
            d2 = t1 - ab_ref[rs, lo:hi]
            blk = jnp.full((_BR, _LANES), j * _KS + k, jnp.int32)
            better = d2 < pm
            pb = jnp.where(better, blk, pb)
            pm = jnp.minimum(d2, pm)
        pmin_ref[rs] = pm
        pblk_ref[rs] = pb
        return _

    lax.fori_loop(0, _NRB, row_block, None)

    @pl.when(j == _NCB - 1)
    def _():
        pm = pmin_ref[...]
        gidx = pblk_ref[...] * _LANES + lax.broadcasted_iota(
            jnp.int32, (_BZ, _LANES), 1)
        m = jnp.min(pm, axis=1)
        amin = jnp.min(jnp.where(pm == m[:, None], gidx, _NE), axis=1)
        idx_ref[...] = amin
        s = jnp.sum(m)
        prev = jnp.where(i == 0, 0.0, loss_ref[0, 0])
        loss_ref[0, 0] = prev + s


_dist_argmin = pl.pallas_call(
    _dist_argmin_body,
    grid=(_NZB, _NCB),
    in_specs=[
        pl.BlockSpec((_BZ, 1), lambda i, j: (i, 0)),
        pl.BlockSpec((_BC,), lambda i, j: (j,)),
        pl.BlockSpec((_BZ, _D), lambda i, j: (i, 0)),
        pl.BlockSpec((_D, _BC), lambda i, j: (0, j)),
    ],
    out_specs=[
        pl.BlockSpec((_BZ,), lambda i, j: (i,)),
        pl.BlockSpec(memory_space=pltpu.SMEM, block_shape=(1, 1),
                     index_map=lambda i, j: (0, 0)),
    ],
    out_shape=[
        jax.ShapeDtypeStruct((_NV,), jnp.int32),
        jax.ShapeDtypeStruct((1, 1), jnp.float32),
    ],
    scratch_shapes=[
        pltpu.VMEM((_BZ, _LANES), jnp.float32),
        pltpu.VMEM((_BZ, _LANES), jnp.int32),
        pltpu.VMEM((_BZ, _BC), jnp.float32),
    ],
)

_NW = 32
_BPW = _NV // _NW
_CH = 128
_NCH = _BPW // _CH

@functools.cache
def _make_gather_rows():
    mesh = plsc.VectorSubcoreMesh(core_axis_name="c", subcore_axis_name="s")

    @functools.partial(
        pl.kernel,
        mesh=mesh,
        out_type=jax.ShapeDtypeStruct((_NV, _D), jnp.float32),
        scratch_types=[
            pltpu.VMEM((_CH,), jnp.int32),
            pltpu.VMEM((_CH, _D), jnp.float32),
            pltpu.SemaphoreType.DMA,
        ],
    )
    def _gather_rows(cb_hbm, idx_hbm, out_hbm, idx_v, rows_v, sem):
        wid = lax.axis_index("s") * 2 + lax.axis_index("c")
        base = wid * _BPW
        for ci in range(_NCH):
            off = base + ci * _CH
            pltpu.sync_copy(idx_hbm.at[pl.ds(off, _CH)], idx_v)
            pltpu.async_copy(cb_hbm.at[idx_v], rows_v, sem).wait()
            pltpu.sync_copy(rows_v, out_hbm.at[pl.ds(off, _CH)])

    return _gather_rows


def kernel(z_feats, codebook):
    a2 = jnp.sum(z_feats * z_feats, axis=1)
    b2 = jnp.sum(codebook * codebook, axis=1)
    idx, loss_sum = _dist_argmin(a2[:, None], b2, z_feats, codebook.T)
    quantized = _make_gather_rows()(codebook, idx)
    loss = loss_sum[0, 0] / jnp.float32(_NV * _D)
    enc = idx.astype(jnp.float32)[:, None]
    return quantized, loss, loss, enc


# speedup vs baseline: 1.5729x; 1.5729x over previous
import functools

import jax
import jax.numpy as jnp
from jax import lax
from jax.experimental import pallas as pl
from jax.experimental.pallas import tpu as pltpu
from jax.experimental.pallas import tpu_sc as plsc

_NE = 8192
_D = 256
_NV = 16384

_BZ = 1024
_BC = 2048
_NZB = _NV // _BZ
_NCB = _NE // _BC


_LANES = 128
_KS = _BC // _LANES
_BR = 128
_NRB = _BZ // _BR


def _dist_argmin_body(a2_ref, b2_ref, zb_ref, cbt_ref, idx_ref, loss_ref,
                      pmin_ref, pblk_ref):
    i = pl.program_id(0)
    j = pl.program_id(1)

    @pl.when(j == 0)
    def _():
        pmin_ref[...] = jnp.full((_BZ, _LANES), jnp.inf, jnp.float32)

    b2 = b2_ref[...]
    cbt = cbt_ref[...]
    for r in range(_NRB):
        rlo, rhi = r * _BR, (r + 1) * _BR
        ab2 = lax.dot_general(
            zb_ref[rlo:rhi, :] * 2.0, cbt, (((1,), (0,)), ((), ())),
            preferred_element_type=jnp.float32)
        a2r = a2_ref[rlo:rhi, :]
        pm = pmin_ref[rlo:rhi, :]
        pb = pblk_ref[rlo:rhi, :]
        for k in range(_KS):
            lo, hi = k * _LANES, (k + 1) * _LANES
            t1 = a2r + b2[lo:hi][None, :]
            d2 = t1 - ab2[:, lo:hi]
            blk = jnp.full((_BR, _LANES), j * _KS + k, jnp.int32)
            better = d2 < pm
            pb = jnp.where(better, blk, pb)
            pm = jnp.minimum(d2, pm)
        pmin_ref[rlo:rhi, :] = pm
        pblk_ref[rlo:rhi, :] = pb

    @pl.when(j == _NCB - 1)
    def _():
        pm = pmin_ref[...]
        gidx = pblk_ref[...] * _LANES + lax.broadcasted_iota(
            jnp.int32, (_BZ, _LANES), 1)
        m = jnp.min(pm, axis=1)
        amin = jnp.min(jnp.where(pm == m[:, None], gidx, _NE), axis=1)
        idx_ref[...] = amin
        s = jnp.sum(m)
        prev = jnp.where(i == 0, 0.0, loss_ref[0, 0])
        loss_ref[0, 0] = prev + s


_dist_argmin = pl.pallas_call(
    _dist_argmin_body,
    grid=(_NZB, _NCB),
    in_specs=[
        pl.BlockSpec((_BZ, 1), lambda i, j: (i, 0)),
        pl.BlockSpec((_BC,), lambda i, j: (j,)),
        pl.BlockSpec((_BZ, _D), lambda i, j: (i, 0)),
        pl.BlockSpec((_D, _BC), lambda i, j: (0, j)),
    ],
    out_specs=[
        pl.BlockSpec((_BZ,), lambda i, j: (i,)),
        pl.BlockSpec(memory_space=pltpu.SMEM, block_shape=(1, 1),
                     index_map=lambda i, j: (0, 0)),
    ],
    out_shape=[
        jax.ShapeDtypeStruct((_NV,), jnp.int32),
        jax.ShapeDtypeStruct((1, 1), jnp.float32),
    ],
    scratch_shapes=[
        pltpu.VMEM((_BZ, _LANES), jnp.float32),
        pltpu.VMEM((_BZ, _LANES), jnp.int32),
    ],
)

_NW = 32
_BPW = _NV // _NW
_CH = 128
_NCH = _BPW // _CH

@functools.cache
def _make_gather_rows():
    mesh = plsc.VectorSubcoreMesh(core_axis_name="c", subcore_axis_name="s")

    @functools.partial(
        pl.kernel,
        mesh=mesh,
        out_type=jax.ShapeDtypeStruct((_NV, _D), jnp.float32),
        scratch_types=[
            pltpu.VMEM((_CH,), jnp.int32),
            pltpu.VMEM((_CH, _D), jnp.float32),
            pltpu.SemaphoreType.DMA,
        ],
    )
    def _gather_rows(cb_hbm, idx_hbm, out_hbm, idx_v, rows_v, sem):
        wid = lax.axis_index("s") * 2 + lax.axis_index("c")
        base = wid * _BPW
        for ci in range(_NCH):
            off = base + ci * _CH
            pltpu.sync_copy(idx_hbm.at[pl.ds(off, _CH)], idx_v)
            pltpu.async_copy(cb_hbm.at[idx_v], rows_v, sem).wait()
            pltpu.sync_copy(rows_v, out_hbm.at[pl.ds(off, _CH)])

    return _gather_rows


def kernel(z_feats, codebook):
    a2 = jnp.sum(z_feats * z_feats, axis=1)
    b2 = jnp.sum(codebook * codebook, axis=1)
    idx, loss_sum = _dist_argmin(a2[:, None], b2, z_feats, codebook.T)
    quantized = _make_gather_rows()(codebook, idx)
    loss = loss_sum[0, 0] / jnp.float32(_NV * _D)
    enc = idx.astype(jnp.float32)[:, None]
    return quantized, loss, loss, enc


# speedup vs baseline: 1.7082x; 1.0861x over previous
import functools

import jax
import jax.numpy as jnp
from jax import lax
from jax.experimental import pallas as pl
from jax.experimental.pallas import tpu as pltpu
from jax.experimental.pallas import tpu_sc as plsc

_NE = 8192
_D = 256
_NV = 16384

_BZ = 1024
_NZB = _NV // _BZ


_LANES = 128
_KS = _NE // _LANES
_BR = 128
_NRB = _BZ // _BR


def _dist_argmin_body(a2_ref, b2_ref, zb_ref, cbt_ref, idx_ref, loss_ref):
    i = pl.program_id(0)
    b2 = b2_ref[...]
    cbt = cbt_ref[...]
    total = None
    for r in range(_NRB):
        rlo, rhi = r * _BR, (r + 1) * _BR
        ab2 = lax.dot_general(
            zb_ref[rlo:rhi, :] * 2.0, cbt, (((1,), (0,)), ((), ())),
            preferred_element_type=jnp.float32)
        a2r = a2_ref[rlo:rhi, :]
        pm = jnp.full((_BR, _LANES), jnp.inf, jnp.float32)
        pb = jnp.zeros((_BR, _LANES), jnp.int32)
        for k in range(_KS):
            lo, hi = k * _LANES, (k + 1) * _LANES
            t1 = a2r + b2[lo:hi][None, :]
            d2 = t1 - ab2[:, lo:hi]
            blk = jnp.full((_BR, _LANES), k, jnp.int32)
            better = d2 < pm
            pb = jnp.where(better, blk, pb)
            pm = jnp.minimum(d2, pm)
        gidx = pb * _LANES + lax.broadcasted_iota(
            jnp.int32, (_BR, _LANES), 1)
        m = jnp.min(pm, axis=1)
        amin = jnp.min(jnp.where(pm == m[:, None], gidx, _NE), axis=1)
        idx_ref[rlo:rhi] = amin
        s = jnp.sum(m)
        total = s if total is None else total + s
    prev = jnp.where(i == 0, 0.0, loss_ref[0, 0])
    loss_ref[0, 0] = prev + total


_dist_argmin = pl.pallas_call(
    _dist_argmin_body,
    grid=(_NZB,),
    in_specs=[
        pl.BlockSpec((_BZ, 1), lambda i: (i, 0)),
        pl.BlockSpec((_NE,), lambda i: (0,)),
        pl.BlockSpec((_BZ, _D), lambda i: (i, 0)),
        pl.BlockSpec((_D, _NE), lambda i: (0, 0)),
    ],
    out_specs=[
        pl.BlockSpec((_BZ,), lambda i: (i,)),
        pl.BlockSpec(memory_space=pltpu.SMEM, block_shape=(1, 1),
                     index_map=lambda i: (0, 0)),
    ],
    out_shape=[
        jax.ShapeDtypeStruct((_NV,), jnp.int32),
        jax.ShapeDtypeStruct((1, 1), jnp.float32),
    ],
)

_NW = 32
_BPW = _NV // _NW
_CH = 128
_NCH = _BPW // _CH

@functools.cache
def _make_gather_rows():
    mesh = plsc.VectorSubcoreMesh(core_axis_name="c", subcore_axis_name="s")

    @functools.partial(
        pl.kernel,
        mesh=mesh,
        out_type=jax.ShapeDtypeStruct((_NV, _D), jnp.float32),
        scratch_types=[
            pltpu.VMEM((_CH,), jnp.int32),
            pltpu.VMEM((_CH, _D), jnp.float32),
            pltpu.SemaphoreType.DMA,
        ],
    )
    def _gather_rows(cb_hbm, idx_hbm, out_hbm, idx_v, rows_v, sem):
        wid = lax.axis_index("s") * 2 + lax.axis_index("c")
        base = wid * _BPW
        for ci in range(_NCH):
            off = base + ci * _CH
            pltpu.sync_copy(idx_hbm.at[pl.ds(off, _CH)], idx_v)
            pltpu.async_copy(cb_hbm.at[idx_v], rows_v, sem).wait()
            pltpu.sync_copy(rows_v, out_hbm.at[pl.ds(off, _CH)])

    return _gather_rows


def kernel(z_feats, codebook):
    a2 = jnp.sum(z_feats * z_feats, axis=1)
    b2 = jnp.sum(codebook * codebook, axis=1)
    idx, loss_sum = _dist_argmin(a2[:, None], b2, z_feats, codebook.T)
    quantized = _make_gather_rows()(codebook, idx)
    loss = loss_sum[0, 0] / jnp.float32(_NV * _D)
    enc = idx.astype(jnp.float32)[:, None]
    return quantized, loss, loss, enc
